# baseline (speedup 1.0000x reference)
import jax
import jax.numpy as jnp
from jax import lax
from jax.experimental import pallas as pl
from jax.experimental.pallas import tpu as pltpu
from jax.experimental.pallas import tpu_sc as plsc

BATCH = 1024
SEQLEN = 200
D = 128
NC = 2
NS = 16
NW = NC * NS
RPW = BATCH // NW
OFFS = (0, 128)
SIZES = (128, SEQLEN - 128)
NBUF = 4
LOOK = 2


def _emb_body(idx_hbm, table_hbm, out_hbm, idx_v, rows_v, gsem, ssem):
    wid = lax.axis_index("s") * NC + lax.axis_index("c")
    row0 = wid * RPW
    pltpu.sync_copy(idx_hbm.at[pl.ds(row0, RPW)], idx_v)

    def start_gathers(row, b):
        for off, size in zip(OFFS, SIZES):
            pltpu.make_async_copy(
                table_hbm.at[idx_v.at[row, pl.ds(off, size)]],
                rows_v.at[b, pl.ds(off, size)],
                gsem.at[b],
            ).start()

    def wait_gathers(b):
        pltpu.make_async_copy(
            table_hbm.at[idx_v.at[0]], rows_v.at[b], gsem.at[b]).wait()

    def start_scatter(row, b):
        pltpu.make_async_copy(
            rows_v.at[b], out_hbm.at[row0 + row], ssem.at[b]).start()

    def wait_scatter(b):
        pltpu.make_async_copy(
            rows_v.at[b], out_hbm.at[row0], ssem.at[b]).wait()

    for b in range(NBUF):
        start_gathers(b, b)

    @pl.loop(0, RPW, step=NBUF)
    def _(r0):
        for b in range(NBUF):
            row = r0 + b
            wait_gathers(b)
            start_scatter(row, b)
            rw = row - LOOK
            bw = (b - LOOK) % NBUF
            rg = rw + NBUF

            @pl.when(jnp.logical_and(rw >= 0, rg < RPW))
            def _():
                wait_scatter(bw)
                start_gathers(rg, bw)

    for b in range(NBUF):
        wait_scatter(b)


_emb_call = pl.kernel(
    _emb_body,
    out_type=jax.ShapeDtypeStruct((BATCH, SEQLEN, D), jnp.float32),
    mesh=plsc.VectorSubcoreMesh(core_axis_name="c", subcore_axis_name="s"),
    scratch_types=[
        pltpu.VMEM((RPW, SEQLEN), jnp.int32),
        pltpu.VMEM((NBUF, SEQLEN, D), jnp.float32),
        pltpu.SemaphoreType.DMA((NBUF,)),
        pltpu.SemaphoreType.DMA((NBUF,)),
    ],
)


def _mask_body(tok_ref, m_ref):
    m_ref[...] = (tok_ref[...] > 0).astype(jnp.int8)


_mask_call = pl.pallas_call(
    _mask_body,
    out_shape=jax.ShapeDtypeStruct((BATCH, SEQLEN), jnp.int8),
)


def kernel(tokenid, table):
    input_emb = _emb_call(tokenid, table)
    mask = _mask_call(tokenid).astype(jnp.bool_)
    return (input_emb, mask)
